# initial kernel scaffold (unmeasured)
import jax
import jax.numpy as jnp
from jax import lax
from jax.experimental import pallas as pl
from jax.experimental.pallas import tpu as pltpu

M = 2048
D = 2048
CHUNK = 256


def kernel(partial, resid, gamma):
    p_bf16 = partial.reshape(M, D).astype(jnp.bfloat16)
    gamma2d = gamma.reshape(1, D)

    def body(p_ref, resid_ref, gamma_ref, out_ref, recv_ref, send_sem, recv_sem):
        my_x = lax.axis_index("x")
        my_y = lax.axis_index("y")
        nbr = (my_x, 1 - my_y)

        barrier = pltpu.get_barrier_semaphore()
        pl.semaphore_signal(
            barrier, inc=1, device_id=nbr, device_id_type=pl.DeviceIdType.MESH
        )
        pl.semaphore_wait(barrier, 1)

        rdma = pltpu.make_async_remote_copy(
            src_ref=p_ref,
            dst_ref=recv_ref,
            send_sem=send_sem,
            recv_sem=recv_sem,
            device_id=nbr,
            device_id_type=pl.DeviceIdType.MESH,
        )
        rdma.start()
        rdma.wait()

        for i in range(M // CHUNK):
            sl = pl.ds(i * CHUNK, CHUNK)
            y = (
                p_ref[sl, :].astype(jnp.float32)
                + recv_ref[sl, :].astype(jnp.float32)
                + resid_ref[sl, :]
            )
            rms = jnp.sqrt(jnp.mean(y * y, axis=-1, keepdims=True) + 1e-6)
            out_ref[sl, :] = y / rms * gamma_ref[0, :][None, :]

    return pl.pallas_call(
        body,
        out_shape=jax.ShapeDtypeStruct((M, D), jnp.float32),
        in_specs=[
            pl.BlockSpec(memory_space=pltpu.VMEM),
            pl.BlockSpec(memory_space=pltpu.VMEM),
            pl.BlockSpec(memory_space=pltpu.VMEM),
        ],
        out_specs=pl.BlockSpec(memory_space=pltpu.VMEM),
        scratch_shapes=[
            pltpu.VMEM((M, D), jnp.bfloat16),
            pltpu.SemaphoreType.DMA,
            pltpu.SemaphoreType.DMA,
        ],
        compiler_params=pltpu.CompilerParams(collective_id=0),
    )(p_bf16, resid, gamma2d)


# baseline (device time: 135615 ns/iter reference)
import jax
import jax.numpy as jnp
from jax import lax
from jax.experimental import pallas as pl
from jax.experimental.pallas import tpu as pltpu

M = 2048
D = 2048
CHUNK = 256


def kernel(partial, resid, gamma):
    p_bf16 = partial.reshape(M, D).astype(jnp.bfloat16)
    gamma2d = gamma.reshape(1, D)

    def body(p_ref, resid_ref, gamma_ref, out_ref, recv_ref, send_sem, recv_sem):
        my_x = lax.axis_index("x")
        my_y = lax.axis_index("y")
        nbr = (my_x, 1 - my_y)

        barrier = pltpu.get_barrier_semaphore()
        pl.semaphore_signal(
            barrier, inc=1, device_id=nbr, device_id_type=pl.DeviceIdType.MESH
        )
        pl.semaphore_wait(barrier, 1)

        rdma = pltpu.make_async_remote_copy(
            src_ref=p_ref,
            dst_ref=recv_ref,
            send_sem=send_sem,
            recv_sem=recv_sem,
            device_id=nbr,
            device_id_type=pl.DeviceIdType.MESH,
        )
        rdma.start()
        rdma.wait()

        def step(i, carry):
            sl = pl.ds(i * CHUNK, CHUNK)
            y = (
                p_ref[sl, :].astype(jnp.float32)
                + recv_ref[sl, :].astype(jnp.float32)
                + resid_ref[sl, :]
            )
            rms = jnp.sqrt(jnp.mean(y * y, axis=-1, keepdims=True) + 1e-6)
            out_ref[sl, :] = y / rms * gamma_ref[0, :][None, :]
            return carry

        lax.fori_loop(0, M // CHUNK, step, 0)

    return pl.pallas_call(
        body,
        out_shape=jax.ShapeDtypeStruct((M, D), jnp.float32),
        in_specs=[
            pl.BlockSpec(memory_space=pltpu.VMEM),
            pl.BlockSpec(memory_space=pltpu.VMEM),
            pl.BlockSpec(memory_space=pltpu.VMEM),
        ],
        out_specs=pl.BlockSpec(memory_space=pltpu.VMEM),
        scratch_shapes=[
            pltpu.VMEM((M, D), jnp.bfloat16),
            pltpu.SemaphoreType.DMA,
            pltpu.SemaphoreType.DMA,
        ],
        compiler_params=pltpu.CompilerParams(
            collective_id=0, vmem_limit_bytes=100 * 1024 * 1024
        ),
    )(p_bf16, resid, gamma2d)


# device time: 93865 ns/iter; 1.4448x vs baseline; 1.4448x over previous
import jax
import jax.numpy as jnp
from jax import lax
from jax.experimental import pallas as pl
from jax.experimental.pallas import tpu as pltpu

M = 2048
D = 2048
HALF = M // 2
C = 8
CH = HALF // C


def kernel(partial, resid, gamma):
    p_bf16 = partial.reshape(M, D).astype(jnp.bfloat16)
    gamma2d = gamma.reshape(1, D)

    def body(
        p_ref,
        resid_ref,
        gamma_ref,
        out_ref,
        other_ref,
        send_y_sems,
        recv_y_sems,
        send_x_sems,
        recv_x_sems,
    ):
        my_x = lax.axis_index("x")
        my_y = lax.axis_index("y")
        nbr_y = (my_x, 1 - my_y)
        nbr_x = (1 - my_x, my_y)

        def off_mine(k):
            return my_x * HALF + k * CH

        def off_other(k):
            return (1 - my_x) * HALF + k * CH

        def rdma_y(k):
            off = off_mine(k)
            return pltpu.make_async_remote_copy(
                src_ref=p_ref.at[pl.ds(off, CH), :],
                dst_ref=other_ref.at[pl.ds(off, CH), :],
                send_sem=send_y_sems.at[k],
                recv_sem=recv_y_sems.at[k],
                device_id=nbr_y,
                device_id_type=pl.DeviceIdType.MESH,
            )

        def rdma_x(k):
            off = off_mine(k)
            return pltpu.make_async_remote_copy(
                src_ref=other_ref.at[pl.ds(off, CH), :],
                dst_ref=other_ref.at[pl.ds(off, CH), :],
                send_sem=send_x_sems.at[k],
                recv_sem=recv_x_sems.at[k],
                device_id=nbr_x,
                device_id_type=pl.DeviceIdType.MESH,
            )

        def compute(off):
            sl = pl.ds(off, CH)
            y = (
                p_ref[sl, :].astype(jnp.float32)
                + other_ref[sl, :].astype(jnp.float32)
                + resid_ref[sl, :]
            )
            rms = jnp.sqrt(jnp.mean(y * y, axis=-1, keepdims=True) + 1e-6)
            out_ref[sl, :] = y / rms * gamma_ref[0, :][None, :]

        barrier = pltpu.get_barrier_semaphore()
        for nbr in (nbr_y, nbr_x):
            pl.semaphore_signal(
                barrier, inc=1, device_id=nbr, device_id_type=pl.DeviceIdType.MESH
            )
        pl.semaphore_wait(barrier, 2)

        def start_y(k, c):
            rdma_y(k).start()
            return c

        lax.fori_loop(0, C, start_y, 0)

        def fwd_and_compute(k, c):
            r = rdma_y(k)
            r.wait_recv()
            rdma_x(k).start()
            compute(off_mine(k))
            return c

        lax.fori_loop(0, C, fwd_and_compute, 0)

        def recv_and_compute(k, c):
            rdma_x(k).wait_recv()
            compute(off_other(k))
            return c

        lax.fori_loop(0, C, recv_and_compute, 0)

        def drain(k, c):
            rdma_y(k).wait_send()
            rdma_x(k).wait_send()
            return c

        lax.fori_loop(0, C, drain, 0)

    return pl.pallas_call(
        body,
        out_shape=jax.ShapeDtypeStruct((M, D), jnp.float32),
        in_specs=[
            pl.BlockSpec(memory_space=pltpu.VMEM),
            pl.BlockSpec(memory_space=pltpu.VMEM),
            pl.BlockSpec(memory_space=pltpu.VMEM),
        ],
        out_specs=pl.BlockSpec(memory_space=pltpu.VMEM),
        scratch_shapes=[
            pltpu.VMEM((M, D), jnp.bfloat16),
            pltpu.SemaphoreType.DMA((C,)),
            pltpu.SemaphoreType.DMA((C,)),
            pltpu.SemaphoreType.DMA((C,)),
            pltpu.SemaphoreType.DMA((C,)),
        ],
        compiler_params=pltpu.CompilerParams(
            collective_id=0, vmem_limit_bytes=100 * 1024 * 1024
        ),
    )(p_bf16, resid, gamma2d)


# device time: 93540 ns/iter; 1.4498x vs baseline; 1.0035x over previous
import jax
import jax.numpy as jnp
from jax import lax
from jax.experimental import pallas as pl
from jax.experimental.pallas import tpu as pltpu

M = 2048
D = 2048
HALF = M // 2
C = 8
CH = HALF // C


def kernel(partial, resid, gamma):
    p_bf16 = partial.reshape(M, D).astype(jnp.bfloat16)
    gamma2d = gamma.reshape(1, D)

    def body(
        p_ref,
        resid_ref,
        gamma_ref,
        out_ref,
        other_ref,
        send_y_sems,
        recv_y_sems,
        send_x_sems,
        recv_x_sems,
    ):
        my_x = lax.axis_index("x")
        my_y = lax.axis_index("y")
        nbr_y = (my_x, 1 - my_y)
        nbr_x = (1 - my_x, my_y)

        def off_mine(k):
            return my_x * HALF + k * CH

        def off_other(k):
            return (1 - my_x) * HALF + k * CH

        def rdma_y(k):
            off = off_mine(k)
            return pltpu.make_async_remote_copy(
                src_ref=p_ref.at[pl.ds(off, CH), :],
                dst_ref=other_ref.at[pl.ds(off, CH), :],
                send_sem=send_y_sems.at[k],
                recv_sem=recv_y_sems.at[k],
                device_id=nbr_y,
                device_id_type=pl.DeviceIdType.MESH,
            )

        def rdma_x(k):
            off = off_mine(k)
            return pltpu.make_async_remote_copy(
                src_ref=other_ref.at[pl.ds(off, CH), :],
                dst_ref=other_ref.at[pl.ds(off, CH), :],
                send_sem=send_x_sems.at[k],
                recv_sem=recv_x_sems.at[k],
                device_id=nbr_x,
                device_id_type=pl.DeviceIdType.MESH,
            )

        def compute(off):
            sl = pl.ds(off, CH)
            y = (
                p_ref[sl, :].astype(jnp.float32)
                + other_ref[sl, :].astype(jnp.float32)
                + resid_ref[sl, :]
            )
            rms = jnp.sqrt(jnp.mean(y * y, axis=-1, keepdims=True) + 1e-6)
            out_ref[sl, :] = y / rms * gamma_ref[0, :][None, :]

        barrier = pltpu.get_barrier_semaphore()
        for nbr in (nbr_y, nbr_x):
            pl.semaphore_signal(
                barrier, inc=1, device_id=nbr, device_id_type=pl.DeviceIdType.MESH
            )
        pl.semaphore_wait(barrier, 2)

        def start_y(k, c):
            rdma_y(k).start()
            return c

        lax.fori_loop(0, C, start_y, 0)

        def fwd_and_compute(k, c):
            r = rdma_y(k)
            r.wait_recv()
            rdma_x(k).start()
            return c

        lax.fori_loop(0, C, fwd_and_compute, 0)

        def recv_and_compute(k, c):
            rdma_x(k).wait_recv()
            return c

        lax.fori_loop(0, C, recv_and_compute, 0)

        def drain(k, c):
            rdma_y(k).wait_send()
            rdma_x(k).wait_send()
            return c

        lax.fori_loop(0, C, drain, 0)

    return pl.pallas_call(
        body,
        out_shape=jax.ShapeDtypeStruct((M, D), jnp.float32),
        in_specs=[
            pl.BlockSpec(memory_space=pltpu.VMEM),
            pl.BlockSpec(memory_space=pltpu.VMEM),
            pl.BlockSpec(memory_space=pltpu.VMEM),
        ],
        out_specs=pl.BlockSpec(memory_space=pltpu.VMEM),
        scratch_shapes=[
            pltpu.VMEM((M, D), jnp.bfloat16),
            pltpu.SemaphoreType.DMA((C,)),
            pltpu.SemaphoreType.DMA((C,)),
            pltpu.SemaphoreType.DMA((C,)),
            pltpu.SemaphoreType.DMA((C,)),
        ],
        compiler_params=pltpu.CompilerParams(
            collective_id=0, vmem_limit_bytes=100 * 1024 * 1024
        ),
    )(p_bf16, resid, gamma2d)


# device time: 88347 ns/iter; 1.5350x vs baseline; 1.0588x over previous
import jax
import jax.numpy as jnp
from jax import lax
from jax.experimental import pallas as pl
from jax.experimental.pallas import tpu as pltpu

M = 2048
D = 2048
HALF = M // 2
CCH = 128
NC = HALF // CCH

SIZES = [64, 288, 320, 288, 64]
OFFS = [0, 64, 352, 672, 960]
NFLOW = len(SIZES)
UNLOCK = [(0, 0), (0, 2), (2, 5), (5, 7), (7, 8)]


def kernel(partial, resid, gamma):
    gamma2d = gamma.reshape(1, D)

    def body(
        p_hbm,
        resid_hbm,
        gamma_ref,
        out_hbm,
        other_ref,
        send_ref,
        pmine_ref,
        pother_ref,
        resid_ref,
        outstage,
        sy_sems,
        ry_sems,
        sx_sems,
        rx_sems,
        pm_sem,
        po_sem,
        re_sem,
        out_sems,
    ):
        my_x = lax.axis_index("x")
        my_y = lax.axis_index("y")
        nbr_y = (my_x, 1 - my_y)
        nbr_x = (1 - my_x, my_y)
        base = my_x * HALF
        obase = (1 - my_x) * HALF

        def rdma_y(k):
            return pltpu.make_async_remote_copy(
                src_ref=send_ref.at[pl.ds(OFFS[k], SIZES[k]), :],
                dst_ref=other_ref.at[pl.ds(base + OFFS[k], SIZES[k]), :],
                send_sem=sy_sems.at[k],
                recv_sem=ry_sems.at[k],
                device_id=nbr_y,
                device_id_type=pl.DeviceIdType.MESH,
            )

        def rdma_x(k):
            return pltpu.make_async_remote_copy(
                src_ref=other_ref.at[pl.ds(base + OFFS[k], SIZES[k]), :],
                dst_ref=other_ref.at[pl.ds(base + OFFS[k], SIZES[k]), :],
                send_sem=sx_sems.at[k],
                recv_sem=rx_sems.at[k],
                device_id=nbr_x,
                device_id_type=pl.DeviceIdType.MESH,
            )

        pm = pltpu.make_async_copy(
            p_hbm.at[0, pl.ds(base, HALF), :], pmine_ref, pm_sem
        )
        pm.start()
        po = pltpu.make_async_copy(
            p_hbm.at[0, pl.ds(obase, HALF), :], pother_ref, po_sem
        )
        po.start()
        re = pltpu.make_async_copy(resid_hbm, resid_ref, re_sem)
        re.start()

        barrier = pltpu.get_barrier_semaphore()
        for nbr in (nbr_y, nbr_x):
            pl.semaphore_signal(
                barrier, inc=1, device_id=nbr, device_id_type=pl.DeviceIdType.MESH
            )
        pl.semaphore_wait(barrier, 2)

        pm.wait()
        for k in range(NFLOW):
            sl = pl.ds(OFFS[k], SIZES[k])
            send_ref[sl, :] = pmine_ref[sl, :].astype(jnp.bfloat16)
            rdma_y(k).start()

        re.wait()

        def compute_chunk(local_ref, loff, g, use_idx):
            slot = lax.rem(use_idx, 2)
            dma = pltpu.make_async_copy(
                outstage.at[slot], out_hbm.at[pl.ds(g, CCH), :], out_sems.at[slot]
            )

            @pl.when(use_idx >= 2)
            def _():
                dma.wait()

            y = (
                local_ref[pl.ds(loff, CCH), :]
                + other_ref[pl.ds(g, CCH), :].astype(jnp.float32)
                + resid_ref[pl.ds(g, CCH), :]
            )
            rms = jnp.sqrt(jnp.mean(y * y, axis=-1, keepdims=True) + 1e-6)
            outstage[slot, :, :] = y / rms * gamma_ref[0, :][None, :]
            dma.start()

        for k in range(NFLOW):
            r = rdma_y(k)
            r.wait_recv()
            rdma_x(k).start()

            def mine(n, c):
                compute_chunk(pmine_ref, n * CCH, base + n * CCH, n)
                return c

            lax.fori_loop(UNLOCK[k][0], UNLOCK[k][1], mine, 0)

        po.wait()
        for k in range(NFLOW):
            rdma_x(k).wait_recv()

            def theirs(n, c):
                compute_chunk(pother_ref, n * CCH, obase + n * CCH, NC + n)
                return c

            lax.fori_loop(UNLOCK[k][0], UNLOCK[k][1], theirs, 0)

        for k in range(NFLOW):
            rdma_y(k).wait_send()
            rdma_x(k).wait_send()
        for slot in range(2):
            pltpu.make_async_copy(
                outstage.at[slot], out_hbm.at[pl.ds(0, CCH), :], out_sems.at[slot]
            ).wait()

    return pl.pallas_call(
        body,
        out_shape=jax.ShapeDtypeStruct((M, D), jnp.float32),
        in_specs=[
            pl.BlockSpec(memory_space=pl.ANY),
            pl.BlockSpec(memory_space=pl.ANY),
            pl.BlockSpec(memory_space=pltpu.VMEM),
        ],
        out_specs=pl.BlockSpec(memory_space=pl.ANY),
        scratch_shapes=[
            pltpu.VMEM((M, D), jnp.bfloat16),
            pltpu.VMEM((HALF, D), jnp.bfloat16),
            pltpu.VMEM((HALF, D), jnp.float32),
            pltpu.VMEM((HALF, D), jnp.float32),
            pltpu.VMEM((M, D), jnp.float32),
            pltpu.VMEM((2, CCH, D), jnp.float32),
            pltpu.SemaphoreType.DMA((NFLOW,)),
            pltpu.SemaphoreType.DMA((NFLOW,)),
            pltpu.SemaphoreType.DMA((NFLOW,)),
            pltpu.SemaphoreType.DMA((NFLOW,)),
            pltpu.SemaphoreType.DMA,
            pltpu.SemaphoreType.DMA,
            pltpu.SemaphoreType.DMA,
            pltpu.SemaphoreType.DMA((2,)),
        ],
        compiler_params=pltpu.CompilerParams(
            collective_id=0, vmem_limit_bytes=100 * 1024 * 1024
        ),
    )(partial, resid, gamma2d)


# device time: 83214 ns/iter; 1.6297x vs baseline; 1.0617x over previous
import jax
import jax.numpy as jnp
from jax import lax
from jax.experimental import pallas as pl
from jax.experimental.pallas import tpu as pltpu

M = 2048
D = 2048
HALF = M // 2
CCH = 128
NC = HALF // CCH

SIZES = [64, 288, 320, 288, 64]
OFFS = [0, 64, 352, 672, 960]
NFLOW = len(SIZES)
UNLOCK = [(0, 0), (0, 2), (2, 5), (5, 7), (7, 8)]


def kernel(partial, resid, gamma):
    gamma2d = gamma.reshape(1, D)

    def body(
        p_hbm,
        resid_hbm,
        gamma_ref,
        out_hbm,
        other_ref,
        send_ref,
        pmine_ref,
        pother_ref,
        resid_ref,
        outstage,
        sy_sems,
        ry_sems,
        sx_sems,
        rx_sems,
        pm_sem,
        po_sem,
        re_sem,
        out_sems,
    ):
        my_x = lax.axis_index("x")
        my_y = lax.axis_index("y")
        nbr_y = (my_x, 1 - my_y)
        nbr_x = (1 - my_x, my_y)
        base = my_x * HALF
        obase = (1 - my_x) * HALF

        def rdma_y(k):
            return pltpu.make_async_remote_copy(
                src_ref=send_ref.at[pl.ds(OFFS[k], SIZES[k]), :],
                dst_ref=other_ref.at[pl.ds(base + OFFS[k], SIZES[k]), :],
                send_sem=sy_sems.at[k],
                recv_sem=ry_sems.at[k],
                device_id=nbr_y,
                device_id_type=pl.DeviceIdType.MESH,
            )

        def rdma_x(k):
            return pltpu.make_async_remote_copy(
                src_ref=other_ref.at[pl.ds(base + OFFS[k], SIZES[k]), :],
                dst_ref=other_ref.at[pl.ds(base + OFFS[k], SIZES[k]), :],
                send_sem=sx_sems.at[k],
                recv_sem=rx_sems.at[k],
                device_id=nbr_x,
                device_id_type=pl.DeviceIdType.MESH,
            )

        pm0 = pltpu.make_async_copy(
            p_hbm.at[0, pl.ds(base, SIZES[0]), :],
            pmine_ref.at[pl.ds(0, SIZES[0]), :],
            pm_sem.at[0],
        )
        pm0.start()
        pm1 = pltpu.make_async_copy(
            p_hbm.at[0, pl.ds(base + SIZES[0], HALF - SIZES[0]), :],
            pmine_ref.at[pl.ds(SIZES[0], HALF - SIZES[0]), :],
            pm_sem.at[1],
        )
        pm1.start()
        re_mine = pltpu.make_async_copy(
            resid_hbm.at[pl.ds(base, HALF), :],
            resid_ref.at[pl.ds(base, HALF), :],
            re_sem.at[0],
        )
        re_mine.start()
        re_other = pltpu.make_async_copy(
            resid_hbm.at[pl.ds(obase, HALF), :],
            resid_ref.at[pl.ds(obase, HALF), :],
            re_sem.at[1],
        )
        re_other.start()
        po = pltpu.make_async_copy(
            p_hbm.at[0, pl.ds(obase, HALF), :], pother_ref, po_sem
        )
        po.start()

        barrier = pltpu.get_barrier_semaphore()
        for nbr in (nbr_y, nbr_x):
            pl.semaphore_signal(
                barrier, inc=1, device_id=nbr, device_id_type=pl.DeviceIdType.MESH
            )
        pl.semaphore_wait(barrier, 2)

        pm0.wait()
        for k in range(NFLOW):
            if k == 1:
                pm1.wait()
            sl = pl.ds(OFFS[k], SIZES[k])
            send_ref[sl, :] = pmine_ref[sl, :].astype(jnp.bfloat16)
            rdma_y(k).start()

        def compute_chunk(local_ref, loff, g, use_idx):
            slot = lax.rem(use_idx, 2)
            dma = pltpu.make_async_copy(
                outstage.at[slot], out_hbm.at[pl.ds(g, CCH), :], out_sems.at[slot]
            )

            @pl.when(use_idx >= 2)
            def _():
                dma.wait()

            y = (
                local_ref[pl.ds(loff, CCH), :]
                + other_ref[pl.ds(g, CCH), :].astype(jnp.float32)
                + resid_ref[pl.ds(g, CCH), :]
            )
            rms = jnp.sqrt(jnp.mean(y * y, axis=-1, keepdims=True) + 1e-6)
            outstage[slot, :, :] = y / rms * gamma_ref[0, :][None, :]
            dma.start()

        for k in range(NFLOW):
            r = rdma_y(k)
            r.wait_recv()
            rdma_x(k).start()
            if k == 1:
                re_mine.wait()

            def mine(n, c):
                compute_chunk(pmine_ref, n * CCH, base + n * CCH, n)
                return c

            lax.fori_loop(UNLOCK[k][0], UNLOCK[k][1], mine, 0)

        for k in range(NFLOW):
            rdma_x(k).wait_recv()
            if k == 1:
                po.wait()
                re_other.wait()

            def theirs(n, c):
                compute_chunk(pother_ref, n * CCH, obase + n * CCH, NC + n)
                return c

            lax.fori_loop(UNLOCK[k][0], UNLOCK[k][1], theirs, 0)

        for k in range(NFLOW):
            rdma_y(k).wait_send()
            rdma_x(k).wait_send()
        for slot in range(2):
            pltpu.make_async_copy(
                outstage.at[slot], out_hbm.at[pl.ds(0, CCH), :], out_sems.at[slot]
            ).wait()

    return pl.pallas_call(
        body,
        out_shape=jax.ShapeDtypeStruct((M, D), jnp.float32),
        in_specs=[
            pl.BlockSpec(memory_space=pl.ANY),
            pl.BlockSpec(memory_space=pl.ANY),
            pl.BlockSpec(memory_space=pltpu.VMEM),
        ],
        out_specs=pl.BlockSpec(memory_space=pl.ANY),
        scratch_shapes=[
            pltpu.VMEM((M, D), jnp.bfloat16),
            pltpu.VMEM((HALF, D), jnp.bfloat16),
            pltpu.VMEM((HALF, D), jnp.float32),
            pltpu.VMEM((HALF, D), jnp.float32),
            pltpu.VMEM((M, D), jnp.float32),
            pltpu.VMEM((2, CCH, D), jnp.float32),
            pltpu.SemaphoreType.DMA((NFLOW,)),
            pltpu.SemaphoreType.DMA((NFLOW,)),
            pltpu.SemaphoreType.DMA((NFLOW,)),
            pltpu.SemaphoreType.DMA((NFLOW,)),
            pltpu.SemaphoreType.DMA((2,)),
            pltpu.SemaphoreType.DMA,
            pltpu.SemaphoreType.DMA((2,)),
            pltpu.SemaphoreType.DMA((2,)),
        ],
        compiler_params=pltpu.CompilerParams(
            collective_id=0, vmem_limit_bytes=100 * 1024 * 1024
        ),
    )(partial, resid, gamma2d)


# device time: 80773 ns/iter; 1.6790x vs baseline; 1.0302x over previous
import jax
import jax.numpy as jnp
from jax import lax
from jax.experimental import pallas as pl
from jax.experimental.pallas import tpu as pltpu

M = 2048
D = 2048
HALF = M // 2
CCH = 128
NC = HALF // CCH

SIZES = [64, 288, 320, 288, 64]
OFFS = [0, 64, 352, 672, 960]
NFLOW = len(SIZES)
UNLOCK = [(0, 0), (0, 2), (2, 5), (5, 7), (7, 8)]


def kernel(partial, resid, gamma):
    gamma2d = gamma.reshape(1, D)

    def body(
        p_hbm,
        resid_hbm,
        gamma_ref,
        out_hbm,
        other_ref,
        send_ref,
        pmine_ref,
        pother_ref,
        resid_ref,
        outstage,
        sy_sems,
        ry_sems,
        sx_sems,
        rx_sems,
        pm_sem,
        po_sem,
        re_sem,
        out_sems,
    ):
        my_x = lax.axis_index("x")
        my_y = lax.axis_index("y")
        nbr_y = (my_x, 1 - my_y)
        nbr_x = (1 - my_x, my_y)
        base = my_x * HALF
        obase = (1 - my_x) * HALF

        def rdma_y(k):
            return pltpu.make_async_remote_copy(
                src_ref=send_ref.at[pl.ds(OFFS[k], SIZES[k]), :],
                dst_ref=other_ref.at[pl.ds(base + OFFS[k], SIZES[k]), :],
                send_sem=sy_sems.at[k],
                recv_sem=ry_sems.at[k],
                device_id=nbr_y,
                device_id_type=pl.DeviceIdType.MESH,
            )

        def rdma_x(k):
            return pltpu.make_async_remote_copy(
                src_ref=other_ref.at[pl.ds(base + OFFS[k], SIZES[k]), :],
                dst_ref=other_ref.at[pl.ds(base + OFFS[k], SIZES[k]), :],
                send_sem=sx_sems.at[k],
                recv_sem=rx_sems.at[k],
                device_id=nbr_x,
                device_id_type=pl.DeviceIdType.MESH,
            )

        pm0 = pltpu.make_async_copy(
            p_hbm.at[0, pl.ds(base, SIZES[0]), :],
            pmine_ref.at[pl.ds(0, SIZES[0]), :],
            pm_sem.at[0],
        )
        pm0.start()
        pm1 = pltpu.make_async_copy(
            p_hbm.at[0, pl.ds(base + SIZES[0], HALF - SIZES[0]), :],
            pmine_ref.at[pl.ds(SIZES[0], HALF - SIZES[0]), :],
            pm_sem.at[1],
        )
        pm1.start()
        re_mine = pltpu.make_async_copy(
            resid_hbm.at[pl.ds(base, HALF), :],
            resid_ref.at[pl.ds(base, HALF), :],
            re_sem.at[0],
        )
        re_mine.start()
        re_other = pltpu.make_async_copy(
            resid_hbm.at[pl.ds(obase, HALF), :],
            resid_ref.at[pl.ds(obase, HALF), :],
            re_sem.at[1],
        )
        po = pltpu.make_async_copy(
            p_hbm.at[0, pl.ds(obase, HALF), :], pother_ref, po_sem
        )

        barrier = pltpu.get_barrier_semaphore()
        for nbr in (nbr_y, nbr_x):
            pl.semaphore_signal(
                barrier, inc=1, device_id=nbr, device_id_type=pl.DeviceIdType.MESH
            )
        pl.semaphore_wait(barrier, 2)

        pm0.wait()
        for k in range(NFLOW):
            if k == 1:
                pm1.wait()
            sl = pl.ds(OFFS[k], SIZES[k])
            send_ref[sl, :] = pmine_ref[sl, :].astype(jnp.bfloat16)
            rdma_y(k).start()

        re_other.start()
        po.start()

        def compute_chunk(local_ref, loff, g, use_idx):
            slot = lax.rem(use_idx, 4)
            dma = pltpu.make_async_copy(
                outstage.at[slot], out_hbm.at[pl.ds(g, CCH), :], out_sems.at[slot]
            )

            @pl.when(use_idx >= 4)
            def _():
                dma.wait()

            y = (
                local_ref[pl.ds(loff, CCH), :]
                + other_ref[pl.ds(g, CCH), :].astype(jnp.float32)
                + resid_ref[pl.ds(g, CCH), :]
            )
            rms = jnp.sqrt(jnp.mean(y * y, axis=-1, keepdims=True) + 1e-6)
            outstage[slot, :, :] = y / rms * gamma_ref[0, :][None, :]
            dma.start()

        for k in range(NFLOW):
            r = rdma_y(k)
            r.wait_recv()
            rdma_x(k).start()
            if k == 1:
                re_mine.wait()

            def mine(n, c):
                compute_chunk(pmine_ref, n * CCH, base + n * CCH, n)
                return c

            lax.fori_loop(UNLOCK[k][0], UNLOCK[k][1], mine, 0)

        for k in range(NFLOW):
            rdma_x(k).wait_recv()
            if k == 1:
                po.wait()
                re_other.wait()

            def theirs(n, c):
                compute_chunk(pother_ref, n * CCH, obase + n * CCH, NC + n)
                return c

            lax.fori_loop(UNLOCK[k][0], UNLOCK[k][1], theirs, 0)

        for k in range(NFLOW):
            rdma_y(k).wait_send()
            rdma_x(k).wait_send()
        for slot in range(4):
            pltpu.make_async_copy(
                outstage.at[slot], out_hbm.at[pl.ds(0, CCH), :], out_sems.at[slot]
            ).wait()

    return pl.pallas_call(
        body,
        out_shape=jax.ShapeDtypeStruct((M, D), jnp.float32),
        in_specs=[
            pl.BlockSpec(memory_space=pl.ANY),
            pl.BlockSpec(memory_space=pl.ANY),
            pl.BlockSpec(memory_space=pltpu.VMEM),
        ],
        out_specs=pl.BlockSpec(memory_space=pl.ANY),
        scratch_shapes=[
            pltpu.VMEM((M, D), jnp.bfloat16),
            pltpu.VMEM((HALF, D), jnp.bfloat16),
            pltpu.VMEM((HALF, D), jnp.float32),
            pltpu.VMEM((HALF, D), jnp.float32),
            pltpu.VMEM((M, D), jnp.float32),
            pltpu.VMEM((4, CCH, D), jnp.float32),
            pltpu.SemaphoreType.DMA((NFLOW,)),
            pltpu.SemaphoreType.DMA((NFLOW,)),
            pltpu.SemaphoreType.DMA((NFLOW,)),
            pltpu.SemaphoreType.DMA((NFLOW,)),
            pltpu.SemaphoreType.DMA((2,)),
            pltpu.SemaphoreType.DMA,
            pltpu.SemaphoreType.DMA((2,)),
            pltpu.SemaphoreType.DMA((4,)),
        ],
        compiler_params=pltpu.CompilerParams(
            collective_id=0, vmem_limit_bytes=100 * 1024 * 1024
        ),
    )(partial, resid, gamma2d)
